# Initial kernel scaffold; baseline (speedup 1.0000x reference)
import jax
import jax.numpy as jnp
from jax.experimental import pallas as pl


def kernel(x, proxies, labels):
    raise NotImplementedError("write your pallas kernel here")


# speedup vs baseline: 3.6296x; 3.6296x over previous
import functools

import jax
import jax.numpy as jnp
from jax import lax
from jax.experimental import pallas as pl
from jax.experimental.pallas import tpu as pltpu
from jax.experimental.pallas import tpu_sc as plsc

N, D, K, C = 4096, 64, 512, 256
BLK = 512
NB = N // BLK

_SC = plsc.get_sparse_core_info()
NW = _SC.num_cores * _SC.num_subcores
BPW = N // NW


def _assign_body(x_ref, p_ref, idx_ref, px_ref):
    xb = x_ref[...]
    p = p_ref[...]
    pn = lax.dot_general(p * p, jnp.ones((1, D), jnp.float32),
                         (((1,), (1,)), ((), ())),
                         preferred_element_type=jnp.float32,
                         precision=lax.Precision.HIGHEST)
    paug = jnp.concatenate([-2.0 * p, pn], axis=1)
    xaug = jnp.concatenate([xb, jnp.ones((BLK, 1), jnp.float32)], axis=1)
    s = lax.dot_general(xaug, paug, (((1,), (1,)), ((), ())),
                        preferred_element_type=jnp.float32,
                        precision=lax.Precision.HIGHEST)
    m = jnp.min(s, axis=1, keepdims=True)
    ks = lax.broadcasted_iota(jnp.int32, (BLK, K), 1)
    idx2 = jnp.min(jnp.where(s == m, ks, K), axis=1, keepdims=True)
    idx_ref[...] = jnp.squeeze(idx2, axis=1)
    oh = (ks == idx2).astype(jnp.float32)
    px_ref[...] = lax.dot_general(oh, p, (((1,), (0,)), ((), ())),
                                  preferred_element_type=jnp.float32,
                                  precision=lax.Precision.HIGHEST)


def _assign(x, proxies):
    return pl.pallas_call(
        _assign_body,
        grid=(NB,),
        in_specs=[pl.BlockSpec((BLK, D), lambda i: (i, 0)),
                  pl.BlockSpec((K, D), lambda i: (0, 0))],
        out_specs=[pl.BlockSpec((BLK,), lambda i: (i,)),
                   pl.BlockSpec((BLK, D), lambda i: (i, 0))],
        out_shape=[jax.ShapeDtypeStruct((N,), jnp.int32),
                   jax.ShapeDtypeStruct((N, D), jnp.float32)],
    )(x, proxies)


def _make_sc_gather():
    mesh = plsc.VectorSubcoreMesh(core_axis_name="c", subcore_axis_name="s")

    @functools.partial(
        pl.kernel,
        mesh=mesh,
        out_type=jax.ShapeDtypeStruct((N, C), jnp.float32),
        scratch_types=[pltpu.VMEM((BPW,), jnp.int32),
                       pltpu.VMEM((BPW, C), jnp.float32),
                       pltpu.SemaphoreType.DMA],
    )
    def gather_k(lab_hbm, idx_hbm, outl_hbm, idx_v, lab_v, sem_l):
        wid = lax.axis_index("s") * _SC.num_cores + lax.axis_index("c")
        base = wid * BPW
        pltpu.sync_copy(idx_hbm.at[pl.ds(base, BPW)], idx_v)
        pltpu.async_copy(lab_hbm.at[idx_v], lab_v, sem_l).wait()
        pltpu.sync_copy(lab_v, outl_hbm.at[pl.ds(base, BPW)])

    return gather_k


_sc_gather = _make_sc_gather()


def kernel(x, proxies, labels):
    idx, px = _assign(x, proxies)
    lx = _sc_gather(labels, idx)
    return x, px, lx
